# Initial kernel scaffold; baseline (speedup 1.0000x reference)
import jax
import jax.numpy as jnp
from jax.experimental import pallas as pl


def kernel(x, edge_index, W0, b0, W1, b1, W2, b2, W3, b3, fc_W, fc_b):
    raise NotImplementedError("write your pallas kernel here")


# speedup vs baseline: 21.5516x; 21.5516x over previous
import jax
import jax.numpy as jnp
from jax import lax
from jax.experimental import pallas as pl
from jax.experimental.pallas import tpu as pltpu
from jax.experimental.pallas import tpu_sc as plsc

_N = 100000
_E = 1600000
_D_IN = 128
_H = 32
_HH = 16
_EC = 128
_NC = 2
_NS = 16
_NPT = 6256
_NPAD = _NPT * _NS
_RPAD = 12544
_EPAD = _RPAD * _EC
_RPT = _RPAD // _NS
_RPW = _RPAD // (_NC * _NS)
_SB = 8
_SBD = 8

_mesh = plsc.VectorSubcoreMesh(core_axis_name="c", subcore_axis_name="s")
_sc_params = pltpu.CompilerParams(use_tc_tiling_on_sc=False)


def _deg_body(dst2, basis, zeros_tab, out0, out1,
              deg_sp, bbuf, dbuf, isem, ssem):
    c = lax.axis_index("c")
    s = lax.axis_index("s")

    rows0 = pl.multiple_of(s * _NPT, 8)
    pltpu.sync_copy(zeros_tab.at[pl.ds(rows0, _NPT)],
                    deg_sp.at[pl.ds(rows0, _NPT)])
    pltpu.sync_copy(basis, bbuf)
    plsc.subcore_barrier()

    w = s * _NC + c
    base = pl.multiple_of(w * _RPW, 8)

    @pl.loop(0, _RPW // _SBD)
    def _sb(i):
        row0 = pl.multiple_of(base + i * _SBD, 8)
        pltpu.async_copy(dst2.at[pl.ds(row0, _SBD)], dbuf, isem).wait()
        adds = [pltpu.async_copy(bbuf, deg_sp.at[dbuf.at[j]], ssem, add=True)
                for j in range(_SBD)]
        for a in adds:
            a.wait()

    plsc.subcore_barrier()

    @pl.when(c == 0)
    def _():
        pltpu.sync_copy(deg_sp.at[pl.ds(rows0, _NPT)],
                        out0.at[pl.ds(rows0, _NPT)])

    @pl.when(c == 1)
    def _():
        pltpu.sync_copy(deg_sp.at[pl.ds(rows0, _NPT)],
                        out1.at[pl.ds(rows0, _NPT)])


_deg_call = pl.kernel(
    _deg_body,
    out_type=(jax.ShapeDtypeStruct((_NPAD, _HH), jnp.float32),
              jax.ShapeDtypeStruct((_NPAD, _HH), jnp.float32)),
    mesh=_mesh,
    scratch_types=[
        pltpu.VMEM_SHARED((_NPAD, _HH), jnp.float32),
        pltpu.VMEM((_EC, _HH), jnp.float32),
        pltpu.VMEM((_SBD, _EC), jnp.int32),
        pltpu.SemaphoreType.DMA,
        pltpu.SemaphoreType.DMA,
    ],
    compiler_params=_sc_params,
)


def _agg_body(zlo, zhi, src2, dst2, outlo, outhi,
              agg_sp, sbuf, dbuf, msg, isem, gsem, ssem):
    c = lax.axis_index("c")
    s = lax.axis_index("s")
    rows0 = pl.multiple_of(s * _NPT, 8)
    base = pl.multiple_of(s * _RPT, 8)

    def run(z_ref, out_ref):
        pltpu.sync_copy(z_ref.at[pl.ds(rows0, _NPT)],
                        agg_sp.at[pl.ds(rows0, _NPT)])
        plsc.subcore_barrier()

        @pl.loop(0, _RPT // _SB)
        def _sb(i):
            row0 = pl.multiple_of(base + i * _SB, 8)
            cs = pltpu.async_copy(src2.at[pl.ds(row0, _SB)], sbuf, isem)
            cd = pltpu.async_copy(dst2.at[pl.ds(row0, _SB)], dbuf, isem)
            cs.wait()
            cd.wait()
            gs = [pltpu.async_copy(z_ref.at[sbuf.at[j]], msg.at[j], gsem)
                  for j in range(_SB)]
            for g in gs:
                g.wait()
            adds = [pltpu.async_copy(msg.at[j], agg_sp.at[dbuf.at[j]],
                                     ssem, add=True)
                    for j in range(_SB)]
            for a in adds:
                a.wait()

        plsc.subcore_barrier()
        pltpu.sync_copy(agg_sp.at[pl.ds(rows0, _NPT)],
                        out_ref.at[pl.ds(rows0, _NPT)])

    @pl.when(c == 0)
    def _():
        run(zlo, outlo)

    @pl.when(c == 1)
    def _():
        run(zhi, outhi)


_agg_call = pl.kernel(
    _agg_body,
    out_type=(jax.ShapeDtypeStruct((_NPAD, _HH), jnp.float32),
              jax.ShapeDtypeStruct((_NPAD, _HH), jnp.float32)),
    mesh=_mesh,
    scratch_types=[
        pltpu.VMEM_SHARED((_NPAD, _HH), jnp.float32),
        pltpu.VMEM((_SB, _EC), jnp.int32),
        pltpu.VMEM((_SB, _EC), jnp.int32),
        pltpu.VMEM((_SB, _EC, _HH), jnp.float32),
        pltpu.SemaphoreType.DMA,
        pltpu.SemaphoreType.DMA,
        pltpu.SemaphoreType.DMA,
    ],
    compiler_params=_sc_params,
)


_BR = 2000
_GPAD = (_NPAD + _BR - 1) // _BR


def _tc_first_body(x_ref, w_ref, d0_ref, d1_ref, zlo_ref, zhi_ref, dinv_ref):
    deg = d0_ref[:, 0:1] + d1_ref[:, 0:1] + 1.0
    dinv = lax.rsqrt(deg)
    z = jnp.dot(x_ref[...], w_ref[...], preferred_element_type=jnp.float32,
                precision=lax.Precision.HIGHEST)
    zs = z * dinv
    zlo_ref[...] = zs[:, :_HH]
    zhi_ref[...] = zs[:, _HH:]
    dinv_ref[...] = dinv


_tc_first = pl.pallas_call(
    _tc_first_body,
    grid=(_GPAD,),
    in_specs=[
        pl.BlockSpec((_BR, _D_IN), lambda i: (i, 0)),
        pl.BlockSpec((_D_IN, _H), lambda i: (0, 0)),
        pl.BlockSpec((_BR, _HH), lambda i: (i, 0)),
        pl.BlockSpec((_BR, _HH), lambda i: (i, 0)),
    ],
    out_specs=(
        pl.BlockSpec((_BR, _HH), lambda i: (i, 0)),
        pl.BlockSpec((_BR, _HH), lambda i: (i, 0)),
        pl.BlockSpec((_BR, 1), lambda i: (i, 0)),
    ),
    out_shape=(
        jax.ShapeDtypeStruct((_NPAD, _HH), jnp.float32),
        jax.ShapeDtypeStruct((_NPAD, _HH), jnp.float32),
        jax.ShapeDtypeStruct((_NPAD, 1), jnp.float32),
    ),
)


def _tc_mid_body(alo_ref, ahi_ref, w_ref, b_ref, dinv_ref, zlo_ref, zhi_ref):
    dinv = dinv_ref[...]
    agg = jnp.concatenate([alo_ref[...], ahi_ref[...]], axis=1)
    h = jnp.maximum(agg * dinv + b_ref[...], 0.0)
    z = jnp.dot(h, w_ref[...], preferred_element_type=jnp.float32,
                precision=lax.Precision.HIGHEST)
    zs = z * dinv
    zlo_ref[...] = zs[:, :_HH]
    zhi_ref[...] = zs[:, _HH:]


_tc_mid = pl.pallas_call(
    _tc_mid_body,
    grid=(_GPAD,),
    in_specs=[
        pl.BlockSpec((_BR, _HH), lambda i: (i, 0)),
        pl.BlockSpec((_BR, _HH), lambda i: (i, 0)),
        pl.BlockSpec((_H, _H), lambda i: (0, 0)),
        pl.BlockSpec((1, _H), lambda i: (0, 0)),
        pl.BlockSpec((_BR, 1), lambda i: (i, 0)),
    ],
    out_specs=(
        pl.BlockSpec((_BR, _HH), lambda i: (i, 0)),
        pl.BlockSpec((_BR, _HH), lambda i: (i, 0)),
    ),
    out_shape=(
        jax.ShapeDtypeStruct((_NPAD, _HH), jnp.float32),
        jax.ShapeDtypeStruct((_NPAD, _HH), jnp.float32),
    ),
)


def _tc_last_body(alo_ref, ahi_ref, b_ref, fcw_ref, fcb_ref, dinv_ref, y_ref):
    dinv = dinv_ref[...]
    agg = jnp.concatenate([alo_ref[...], ahi_ref[...]], axis=1)
    h = jnp.maximum(agg * dinv + b_ref[...], 0.0)
    y_ref[...] = jnp.sum(h * fcw_ref[...], axis=1, keepdims=True) + fcb_ref[...]


_tc_last = pl.pallas_call(
    _tc_last_body,
    grid=(_N // _BR,),
    in_specs=[
        pl.BlockSpec((_BR, _HH), lambda i: (i, 0)),
        pl.BlockSpec((_BR, _HH), lambda i: (i, 0)),
        pl.BlockSpec((1, _H), lambda i: (0, 0)),
        pl.BlockSpec((1, _H), lambda i: (0, 0)),
        pl.BlockSpec((1, 1), lambda i: (0, 0)),
        pl.BlockSpec((_BR, 1), lambda i: (i, 0)),
    ],
    out_specs=pl.BlockSpec((_BR, 1), lambda i: (i, 0)),
    out_shape=jax.ShapeDtypeStruct((_N, 1), jnp.float32),
)


def kernel(x, edge_index, W0, b0, W1, b1, W2, b2, W3, b3, fc_W, fc_b):
    pad = _EPAD - _E
    src2 = jnp.concatenate(
        [edge_index[0], jnp.zeros((pad,), jnp.int32)]).reshape(_RPAD, _EC)
    dst2 = jnp.concatenate(
        [edge_index[1], jnp.full((pad,), _N, jnp.int32)]).reshape(_RPAD, _EC)
    basis = jnp.zeros((_EC, _HH), jnp.float32).at[:, 0].set(1.0)
    zeros_tab = jnp.zeros((_NPAD, _HH), jnp.float32)

    d0, d1 = _deg_call(dst2, basis, zeros_tab)
    zlo, zhi, dinv = _tc_first(x, W0, d0, d1)
    for (W, b) in ((W1, b0), (W2, b1), (W3, b2)):
        alo, ahi = _agg_call(zlo, zhi, src2, dst2)
        zlo, zhi = _tc_mid(alo, ahi, W, b.reshape(1, _H), dinv)
    alo, ahi = _agg_call(zlo, zhi, src2, dst2)
    y = _tc_last(alo, ahi, b3.reshape(1, _H), fc_W.reshape(1, _H),
                 fc_b.reshape(1, 1), dinv)
    return y[:, 0]


# speedup vs baseline: 30.8937x; 1.4335x over previous
import jax
import jax.numpy as jnp
from jax import lax
from jax.experimental import pallas as pl
from jax.experimental.pallas import tpu as pltpu
from jax.experimental.pallas import tpu_sc as plsc

_N = 100000
_E = 1600000
_D_IN = 128
_H = 32
_HH = 16
_EC = 128
_NC = 2
_NS = 16
_NPT = 6256
_NPAD = _NPT * _NS
_RPAD = 12544
_EPAD = _RPAD * _EC
_RPT = _RPAD // _NS
_RPW = _RPAD // (_NC * _NS)
_SB = 8
_SBD = 8

_mesh = plsc.VectorSubcoreMesh(core_axis_name="c", subcore_axis_name="s")
_sc_params = pltpu.CompilerParams(use_tc_tiling_on_sc=False)


def _deg_body(dst2, basis, zeros_tab, out0, out1,
              deg_sp, bbuf, dbuf, isem, ssem):
    c = lax.axis_index("c")
    s = lax.axis_index("s")

    rows0 = pl.multiple_of(s * _NPT, 8)
    pltpu.sync_copy(zeros_tab.at[pl.ds(rows0, _NPT)],
                    deg_sp.at[pl.ds(rows0, _NPT)])
    pltpu.sync_copy(basis, bbuf)
    plsc.subcore_barrier()

    w = s * _NC + c
    base = pl.multiple_of(w * _RPW, 8)

    @pl.loop(0, _RPW // _SBD)
    def _sb(i):
        row0 = pl.multiple_of(base + i * _SBD, 8)
        pltpu.async_copy(dst2.at[pl.ds(row0, _SBD)], dbuf, isem).wait()
        adds = [pltpu.async_copy(bbuf, deg_sp.at[dbuf.at[j]], ssem, add=True)
                for j in range(_SBD)]
        for a in adds:
            a.wait()

    plsc.subcore_barrier()

    @pl.when(c == 0)
    def _():
        pltpu.sync_copy(deg_sp.at[pl.ds(rows0, _NPT)],
                        out0.at[pl.ds(rows0, _NPT)])

    @pl.when(c == 1)
    def _():
        pltpu.sync_copy(deg_sp.at[pl.ds(rows0, _NPT)],
                        out1.at[pl.ds(rows0, _NPT)])


_deg_call = pl.kernel(
    _deg_body,
    out_type=(jax.ShapeDtypeStruct((_NPAD, _HH), jnp.float32),
              jax.ShapeDtypeStruct((_NPAD, _HH), jnp.float32)),
    mesh=_mesh,
    scratch_types=[
        pltpu.VMEM_SHARED((_NPAD, _HH), jnp.float32),
        pltpu.VMEM((_EC, _HH), jnp.float32),
        pltpu.VMEM((_SBD, _EC), jnp.int32),
        pltpu.SemaphoreType.DMA,
        pltpu.SemaphoreType.DMA,
    ],
    compiler_params=_sc_params,
)


def _agg_body(zlo, zhi, src2, dst2, outlo, outhi,
              agg_sp, sbuf, dbuf, msg, isem, gsem, ssem):
    c = lax.axis_index("c")
    s = lax.axis_index("s")
    rows0 = pl.multiple_of(s * _NPT, 8)
    base = pl.multiple_of(s * _RPT, 8)

    def run(z_ref, out_ref):
        pltpu.sync_copy(z_ref.at[pl.ds(rows0, _NPT)],
                        agg_sp.at[pl.ds(rows0, _NPT)])
        plsc.subcore_barrier()

        @pl.loop(0, _RPT // _SB)
        def _sb(i):
            row0 = pl.multiple_of(base + i * _SB, 8)
            cs = pltpu.async_copy(src2.at[pl.ds(row0, _SB)], sbuf, isem)
            cd = pltpu.async_copy(dst2.at[pl.ds(row0, _SB)], dbuf, isem)
            cs.wait()
            cd.wait()
            gs = [pltpu.async_copy(z_ref.at[sbuf.at[j]], msg.at[j], gsem)
                  for j in range(_SB)]
            for g in gs:
                g.wait()
            adds = [pltpu.async_copy(msg.at[j], agg_sp.at[dbuf.at[j]],
                                     ssem, add=True)
                    for j in range(_SB)]
            for a in adds:
                a.wait()

        plsc.subcore_barrier()
        pltpu.sync_copy(agg_sp.at[pl.ds(rows0, _NPT)],
                        out_ref.at[pl.ds(rows0, _NPT)])

    @pl.when(c == 0)
    def _():
        run(zlo, outlo)

    @pl.when(c == 1)
    def _():
        run(zhi, outhi)


_agg_call = pl.kernel(
    _agg_body,
    out_type=(jax.ShapeDtypeStruct((_NPAD, _HH), jnp.float32),
              jax.ShapeDtypeStruct((_NPAD, _HH), jnp.float32)),
    mesh=_mesh,
    scratch_types=[
        pltpu.VMEM_SHARED((_NPAD, _HH), jnp.float32),
        pltpu.VMEM((_SB, _EC), jnp.int32),
        pltpu.VMEM((_SB, _EC), jnp.int32),
        pltpu.VMEM((_SB, _EC, _HH), jnp.float32),
        pltpu.SemaphoreType.DMA,
        pltpu.SemaphoreType.DMA,
        pltpu.SemaphoreType.DMA,
    ],
    compiler_params=_sc_params,
)


_FR = _NPAD * _HH // 128
_XR = _N * _D_IN // 1024
_BN = 2048
_BF = _BN * _HH // 128
_GPAD = (_FR + _BF - 1) // _BF
_HP = lax.Precision.HIGHEST


def _tc_first_body(x_ref, klo_ref, khi_ref, d0_ref, d1_ref, s_ref,
                   zlo_ref, zhi_ref, dinv_ref):
    dband = jnp.dot(d0_ref[...] + d1_ref[...], s_ref[...],
                    preferred_element_type=jnp.float32, precision=_HP)
    dinv = lax.rsqrt(dband + 1.0)
    xb = x_ref[...]
    zlo_ref[...] = jnp.dot(xb, klo_ref[...],
                           preferred_element_type=jnp.float32,
                           precision=_HP) * dinv
    zhi_ref[...] = jnp.dot(xb, khi_ref[...],
                           preferred_element_type=jnp.float32,
                           precision=_HP) * dinv
    dinv_ref[...] = dinv


_tc_first = pl.pallas_call(
    _tc_first_body,
    grid=(_GPAD,),
    in_specs=[
        pl.BlockSpec((_BF, 1024), lambda i: (i, 0)),
        pl.BlockSpec((1024, 128), lambda i: (0, 0)),
        pl.BlockSpec((1024, 128), lambda i: (0, 0)),
        pl.BlockSpec((_BF, 128), lambda i: (i, 0)),
        pl.BlockSpec((_BF, 128), lambda i: (i, 0)),
        pl.BlockSpec((128, 128), lambda i: (0, 0)),
    ],
    out_specs=(
        pl.BlockSpec((_BF, 128), lambda i: (i, 0)),
        pl.BlockSpec((_BF, 128), lambda i: (i, 0)),
        pl.BlockSpec((_BF, 128), lambda i: (i, 0)),
    ),
    out_shape=(
        jax.ShapeDtypeStruct((_FR, 128), jnp.float32),
        jax.ShapeDtypeStruct((_FR, 128), jnp.float32),
        jax.ShapeDtypeStruct((_FR, 128), jnp.float32),
    ),
)


def _tc_mid_body(alo_ref, ahi_ref, kll_ref, khl_ref, klh_ref, khh_ref,
                 blo_ref, bhi_ref, dinv_ref, zlo_ref, zhi_ref):
    dinv = dinv_ref[...]
    hlo = jnp.maximum(alo_ref[...] * dinv + blo_ref[...], 0.0)
    hhi = jnp.maximum(ahi_ref[...] * dinv + bhi_ref[...], 0.0)
    zlo = (jnp.dot(hlo, kll_ref[...], preferred_element_type=jnp.float32,
                   precision=_HP)
           + jnp.dot(hhi, khl_ref[...], preferred_element_type=jnp.float32,
                     precision=_HP))
    zhi = (jnp.dot(hlo, klh_ref[...], preferred_element_type=jnp.float32,
                   precision=_HP)
           + jnp.dot(hhi, khh_ref[...], preferred_element_type=jnp.float32,
                     precision=_HP))
    zlo_ref[...] = zlo * dinv
    zhi_ref[...] = zhi * dinv


_tc_mid = pl.pallas_call(
    _tc_mid_body,
    grid=(_GPAD,),
    in_specs=[
        pl.BlockSpec((_BF, 128), lambda i: (i, 0)),
        pl.BlockSpec((_BF, 128), lambda i: (i, 0)),
        pl.BlockSpec((128, 128), lambda i: (0, 0)),
        pl.BlockSpec((128, 128), lambda i: (0, 0)),
        pl.BlockSpec((128, 128), lambda i: (0, 0)),
        pl.BlockSpec((128, 128), lambda i: (0, 0)),
        pl.BlockSpec((1, 128), lambda i: (0, 0)),
        pl.BlockSpec((1, 128), lambda i: (0, 0)),
        pl.BlockSpec((_BF, 128), lambda i: (i, 0)),
    ],
    out_specs=(
        pl.BlockSpec((_BF, 128), lambda i: (i, 0)),
        pl.BlockSpec((_BF, 128), lambda i: (i, 0)),
    ),
    out_shape=(
        jax.ShapeDtypeStruct((_FR, 128), jnp.float32),
        jax.ShapeDtypeStruct((_FR, 128), jnp.float32),
    ),
)


def _tc_last_body(alo_ref, ahi_ref, blo_ref, bhi_ref, flo_ref, fhi_ref,
                  ssum_ref, fcb_ref, dinv_ref, y_ref):
    dinv = dinv_ref[...]
    hlo = jnp.maximum(alo_ref[...] * dinv + blo_ref[...], 0.0)
    hhi = jnp.maximum(ahi_ref[...] * dinv + bhi_ref[...], 0.0)
    t = hlo * flo_ref[...] + hhi * fhi_ref[...]
    y_ref[...] = jnp.dot(t, ssum_ref[...], preferred_element_type=jnp.float32,
                         precision=_HP) + fcb_ref[...]


_tc_last = pl.pallas_call(
    _tc_last_body,
    grid=(_GPAD,),
    in_specs=[
        pl.BlockSpec((_BF, 128), lambda i: (i, 0)),
        pl.BlockSpec((_BF, 128), lambda i: (i, 0)),
        pl.BlockSpec((1, 128), lambda i: (0, 0)),
        pl.BlockSpec((1, 128), lambda i: (0, 0)),
        pl.BlockSpec((1, 128), lambda i: (0, 0)),
        pl.BlockSpec((1, 128), lambda i: (0, 0)),
        pl.BlockSpec((128, 8), lambda i: (0, 0)),
        pl.BlockSpec((1, 8), lambda i: (0, 0)),
        pl.BlockSpec((_BF, 128), lambda i: (i, 0)),
    ],
    out_specs=pl.BlockSpec((_BF, 8), lambda i: (i, 0)),
    out_shape=jax.ShapeDtypeStruct((_FR, 8), jnp.float32),
)


def kernel(x, edge_index, W0, b0, W1, b1, W2, b2, W3, b3, fc_W, fc_b):
    pad = _EPAD - _E
    src2 = jnp.concatenate(
        [edge_index[0], jnp.zeros((pad,), jnp.int32)]).reshape(_RPAD, _EC)
    dst2 = jnp.concatenate(
        [edge_index[1], jnp.full((pad,), _N, jnp.int32)]).reshape(_RPAD, _EC)
    basis = jnp.zeros((_EC, _HH), jnp.float32).at[:, 0].set(1.0)
    zeros_tab = jnp.zeros((_NPAD, _HH), jnp.float32)

    eye8 = jnp.eye(8, dtype=jnp.float32)
    s_spread = jnp.kron(eye8, jnp.zeros((16, 16), jnp.float32)
                        .at[0, :].set(1.0))
    s_sum = jnp.kron(eye8, jnp.ones((16, 1), jnp.float32))

    def flat(t):
        return t.reshape(_FR, 128)

    def unflat(t):
        return t.reshape(_NPAD, _HH)

    d0, d1 = _deg_call(dst2, basis, zeros_tab)
    zlo_f, zhi_f, dinv_f = _tc_first(
        x.reshape(_XR, 1024),
        jnp.kron(eye8, W0[:, :_HH]), jnp.kron(eye8, W0[:, _HH:]),
        flat(d0), flat(d1), s_spread)
    for (W, b) in ((W1, b0), (W2, b1), (W3, b2)):
        alo, ahi = _agg_call(unflat(zlo_f), unflat(zhi_f), src2, dst2)
        zlo_f, zhi_f = _tc_mid(
            flat(alo), flat(ahi),
            jnp.kron(eye8, W[:_HH, :_HH]), jnp.kron(eye8, W[_HH:, :_HH]),
            jnp.kron(eye8, W[:_HH, _HH:]), jnp.kron(eye8, W[_HH:, _HH:]),
            jnp.tile(b[:_HH], 8).reshape(1, 128),
            jnp.tile(b[_HH:], 8).reshape(1, 128),
            dinv_f)
    alo, ahi = _agg_call(unflat(zlo_f), unflat(zhi_f), src2, dst2)
    y8 = _tc_last(
        flat(alo), flat(ahi),
        jnp.tile(b3[:_HH], 8).reshape(1, 128),
        jnp.tile(b3[_HH:], 8).reshape(1, 128),
        jnp.tile(fc_W[:_HH, 0], 8).reshape(1, 128),
        jnp.tile(fc_W[_HH:, 0], 8).reshape(1, 128),
        s_sum, jnp.tile(fc_b, 8).reshape(1, 8), dinv_f)
    return y8.reshape(_NPAD)[:_N]
